# baseline (speedup 1.0000x reference)
import functools

import jax
import jax.numpy as jnp
from jax import lax
from jax.experimental import pallas as pl
from jax.experimental.pallas import tpu as pltpu
from jax.experimental.pallas import tpu_sc as plsc

BN = 256
EBN = 1024
KNN = 20
_UNROLL = False
_INTERPRET = False

_NEG = float("-inf")
_HI = lax.Precision.HIGHEST


def _emb_kernel(occ_ref, lev_ref, oct_ref, t032, t132, t232, t0128, t1128,
                t2128, t0512, t1512, t2512, e32_ref, e128_ref, e512_ref):
  i257 = lax.broadcasted_iota(jnp.int32, (BN, 257), 1)
  i19 = lax.broadcasted_iota(jnp.int32, (BN, 19), 1)
  i9 = lax.broadcasted_iota(jnp.int32, (BN, 9), 1)
  f32 = jnp.float32
  for j in range(4):
    oc = occ_ref[0, :, j]
    oh = (oc[:, None] == i257).astype(f32)
    e32_ref[0, :, j * 8:j * 8 + 6] = jnp.dot(
        oh, t032[...], preferred_element_type=f32, precision=_HI)
    e128_ref[0, :, j * 32:j * 32 + 30] = jnp.dot(
        oh, t0128[...], preferred_element_type=f32, precision=_HI)
    e512_ref[0, :, j * 128:j * 128 + 126] = jnp.dot(
        oh, t0512[...], preferred_element_type=f32, precision=_HI)
    lv = lev_ref[0, :, j]
    ohl = (lv[:, None] == i19).astype(f32)
    e32_ref[0, :, j * 8 + 6:j * 8 + 7] = jnp.dot(
        ohl, t132[...], preferred_element_type=f32, precision=_HI)
    e128_ref[0, :, j * 32 + 30:j * 32 + 31] = jnp.dot(
        ohl, t1128[...], preferred_element_type=f32, precision=_HI)
    e512_ref[0, :, j * 128 + 126:j * 128 + 127] = jnp.dot(
        ohl, t1512[...], preferred_element_type=f32, precision=_HI)
    ot = oct_ref[0, :, j]
    oho = (ot[:, None] == i9).astype(f32)
    e32_ref[0, :, j * 8 + 7:j * 8 + 8] = jnp.dot(
        oho, t232[...], preferred_element_type=f32, precision=_HI)
    e128_ref[0, :, j * 32 + 31:j * 32 + 32] = jnp.dot(
        oho, t2128[...], preferred_element_type=f32, precision=_HI)
    e512_ref[0, :, j * 128 + 127:j * 128 + 128] = jnp.dot(
        oho, t2512[...], preferred_element_type=f32, precision=_HI)


_SC_W = 128


def _sc_emb_gather(occF, levF, octF, tf32, tf128, tf512, nidx):
  f32 = jnp.float32
  mesh = plsc.VectorSubcoreMesh(core_axis_name="c", subcore_axis_name="s")

  @functools.partial(
      pl.kernel,
      out_type=[
          jax.ShapeDtypeStruct((nidx, 128), f32),
          jax.ShapeDtypeStruct((nidx, 128), f32),
          jax.ShapeDtypeStruct((nidx, 128), f32),
      ],
      mesh=mesh,
      scratch_types=[pltpu.VMEM((1, _SC_W), jnp.int32)],
  )
  def k(occ_hbm, lev_hbm, oct_hbm, t32_hbm, t128_hbm, t512_hbm,
        o32_hbm, o128_hbm, o512_hbm, idx_s):
    def body(occ_v, lev_v, oct_v, o32_v, o128_v, o512_v):
      @pl.loop(0, _SC_W, step=16)
      def _(j):
        sl = (0, pl.ds(j, 16))
        idx_s[sl] = occ_v[sl] * 171 + lev_v[sl] * 9 + oct_v[sl]

      pltpu.sync_copy(t32_hbm.at[idx_s.at[0]], o32_v)
      pltpu.sync_copy(t128_hbm.at[idx_s.at[0]], o128_v)
      pltpu.sync_copy(t512_hbm.at[idx_s.at[0]], o512_v)

    pltpu.emit_pipeline(
        body,
        grid=(nidx // _SC_W,),
        in_specs=[pl.BlockSpec((1, _SC_W), lambda i: (0, i))] * 3,
        out_specs=[
            pl.BlockSpec((_SC_W, 128), lambda i: (i, 0)),
            pl.BlockSpec((_SC_W, 128), lambda i: (i, 0)),
            pl.BlockSpec((_SC_W, 128), lambda i: (i, 0)),
        ],
        core_axis_name=("c", "s"),
        dimension_semantics=(pltpu.PARALLEL,),
    )(occ_hbm, lev_hbm, oct_hbm, o32_hbm, o128_hbm, o512_hbm)

  return k(occF, levF, octF, tf32, tf128, tf512)


def _fuse_tables(e0, e1, e2):
  a = jnp.repeat(e0, 19 * 9, axis=0)
  b = jnp.tile(jnp.repeat(e1, 9, axis=0), (257, 1))
  c = jnp.tile(e2, (257 * 19, 1))
  cols = [a, b, c]
  pad = 128 - (e0.shape[1] + 2)
  if pad:
    cols.append(jnp.zeros((257 * 19 * 9, pad), e0.dtype))
  return jnp.concatenate(cols, axis=1)


def _pairwise_block(x_ref, i, L, bn):
  f32 = jnp.float32
  x = x_ref[0]
  xblk = x_ref[0, pl.ds(i * bn, bn), :]
  g = lax.dot_general(xblk, x, (((1,), (1,)), ((), ())),
                      preferred_element_type=f32)
  xxall = jnp.sum(x * x, axis=1)
  xxblk = jnp.sum(xblk * xblk, axis=1)
  inner = -2.0 * g
  return (-xxblk[:, None]) - inner - xxall[None, :]


def _topk_passes(pd_s, body_fn, L, bn):
  iota = lax.broadcasted_iota(jnp.int32, (bn, L), 1)

  def one_pass(t, carry):
    pdv = pd_s[...]
    vmax = jnp.max(pdv, axis=1, keepdims=True)
    eq = pdv == vmax
    cand = jnp.where(eq, iota, L)
    pos = jnp.min(cand, axis=1, keepdims=True)
    h = iota == pos
    pd_s[...] = jnp.where(h, _NEG, pdv)
    body_fn(h.astype(jnp.float32))
    return carry

  if _UNROLL:
    for t in range(KNN):
      one_pass(t, 0)
  else:
    lax.fori_loop(0, KNN, one_pass, 0, unroll=False)


def _edge_exact_kernel(x_ref, wt_ref, ymax_ref, part_ref,
                       pd_s, s_s, q_s, m_s, *, L, O, C):
  f32 = jnp.float32
  i = pl.program_id(1)

  @pl.when(i == 0)
  def _():
    part_ref[...] = jnp.zeros(part_ref.shape, f32)

  x = x_ref[0]
  xblk = x_ref[0, pl.ds(i * EBN, EBN), :]
  pd_s[...] = _pairwise_block(x_ref, i, L, EBN)
  s_s[...] = jnp.zeros((EBN, O), f32)
  q_s[...] = jnp.zeros((EBN, O), f32)
  m_s[...] = jnp.full((EBN, O), _NEG, f32)

  def body(hf):
    feat = jnp.dot(hf, x, preferred_element_type=f32, precision=_HI)
    v = jnp.concatenate([feat - xblk, xblk], axis=1)
    y = jnp.dot(v, wt_ref[...], preferred_element_type=f32)
    s_s[...] += y
    q_s[...] += y * y
    m_s[...] = jnp.maximum(m_s[...], y)

  _topk_passes(pd_s, body, L, EBN)

  ymax_ref[0] = m_s[...]
  zrow = jnp.zeros((1, O), f32)
  upd = jnp.concatenate([
      jnp.sum(s_s[...], axis=0)[None, :],
      jnp.sum(q_s[...], axis=0)[None, :],
      zrow, zrow, zrow, zrow, zrow, zrow], axis=0)
  part_ref[0] += upd


def _edge_fact_kernel(x_ref, wat_ref, wdt_ref, ymax_ref, part_ref,
                      z_s, w_s, pd_s, s_s, q_s, m_s, *, L, O):
  f32 = jnp.float32
  i = pl.program_id(1)

  @pl.when(i == 0)
  def _():
    x0 = x_ref[0]
    z_s[...] = jnp.dot(x0, wat_ref[...], preferred_element_type=f32)
    w_s[...] = jnp.dot(x0, wdt_ref[...], preferred_element_type=f32)
    part_ref[...] = jnp.zeros(part_ref.shape, f32)

  pd_s[...] = _pairwise_block(x_ref, i, L, EBN)
  s_s[...] = jnp.zeros((EBN, O), f32)
  q_s[...] = jnp.zeros((EBN, O), f32)
  m_s[...] = jnp.full((EBN, O), _NEG, f32)

  def body(hf):
    gi = jnp.dot(hf, z_s[...], preferred_element_type=f32)
    s_s[...] += gi
    q_s[...] += gi * gi
    m_s[...] = jnp.maximum(m_s[...], gi)

  _topk_passes(pd_s, body, L, EBN)

  wblk = w_s[pl.ds(i * EBN, EBN), :]
  sv = s_s[...]
  ymax_ref[0] = m_s[...] + wblk
  zrow = jnp.zeros((1, O), f32)
  upd = jnp.concatenate([
      jnp.sum(sv, axis=0)[None, :],
      jnp.sum(q_s[...], axis=0)[None, :],
      jnp.sum(wblk * sv, axis=0)[None, :],
      jnp.sum(wblk, axis=0)[None, :],
      jnp.sum(wblk * wblk, axis=0)[None, :],
      zrow, zrow, zrow], axis=0)
  part_ref[0] += upd


def _finalize_kernel(ymax_ref, part_ref, gb_ref, emb_ref, xcat_ref, *,
                     cnt, O, fact):
  ps = part_ref[0] + part_ref[1]
  if fact:
    mu = (ps[0] + float(KNN) * ps[3]) / cnt
    e2 = (ps[1] + 2.0 * ps[2] + float(KNN) * ps[4]) / cnt
  else:
    mu = ps[0] / cnt
    e2 = ps[1] / cnt
  var = e2 - mu * mu
  sq = jnp.sqrt(var + 1e-5)
  v = (ymax_ref[0] - mu[None, :]) / sq[None, :] * gb_ref[0][None, :] \
      + gb_ref[1][None, :]
  v = jnp.where(v >= 0, v, 0.2 * v)
  xcat_ref[0, :, :O] = v
  xcat_ref[0, :, O:] = emb_ref[0]


def _head_kernel(x1_ref, x3_ref, x5_ref, wct_ref, y_ref, part_ref):
  f32 = jnp.float32
  i = pl.program_id(1)

  @pl.when(i == 0)
  def _():
    part_ref[...] = jnp.zeros(part_ref.shape, f32)

  xcat = jnp.concatenate([x1_ref[0], x3_ref[0], x5_ref[0]], axis=1)
  y = jnp.dot(xcat, wct_ref[...], preferred_element_type=f32)
  y_ref[0] = y
  zrow = jnp.zeros((1, 512), f32)
  upd = jnp.concatenate([
      jnp.sum(y, axis=0)[None, :],
      jnp.sum(y * y, axis=0)[None, :],
      zrow, zrow, zrow, zrow, zrow, zrow], axis=0)
  part_ref[0] += upd


def _pool_kernel(y_ref, part_ref, gb_ref, pool_ref, *, cnt):
  i = pl.program_id(1)
  ps = part_ref[0] + part_ref[1]
  mu = ps[0] / cnt
  var = ps[1] / cnt - mu * mu
  sq = jnp.sqrt(var + 1e-5)
  yn = (y_ref[0] - mu[None, :]) / sq[None, :] * gb_ref[0][None, :] \
      + gb_ref[1][None, :]
  yn = jnp.where(yn >= 0, yn, 0.2 * yn)

  @pl.when(i == 0)
  def _():
    pool_ref[...] = jnp.full(pool_ref.shape, _NEG, jnp.float32)
    pool_ref[0, 0, :] = jnp.zeros((512,), jnp.float32)

  pool_ref[0, 0, :] += jnp.sum(yn, axis=0)
  pool_ref[0, 1, :] = jnp.maximum(pool_ref[0, 1, :], jnp.max(yn, axis=0))


def _out_kernel(x5_ref, pool_ref, wmt_ref, bm_ref, o_ref, *, L):
  f32 = jnp.float32
  avg = pool_ref[0, 0, :] / float(L)
  mx = pool_ref[0, 1, :]
  big = jnp.concatenate([
      x5_ref[0],
      jnp.broadcast_to(avg[None, :], (BN, 512)),
      jnp.broadcast_to(mx[None, :], (BN, 512))], axis=1)
  o_ref[0] = jnp.dot(big, wmt_ref[...], preferred_element_type=f32) \
      + bm_ref[...]


_CP = pltpu.CompilerParams(dimension_semantics=("parallel", "arbitrary"))


def _edge_layer_exact(x, wt, B, L, O):
  C = x.shape[-1]
  return pl.pallas_call(
      functools.partial(_edge_exact_kernel, L=L, O=O, C=C),
      grid=(B, L // EBN),
      in_specs=[
          pl.BlockSpec((1, L, C), lambda b, i: (b, 0, 0)),
          pl.BlockSpec((2 * C, O), lambda b, i: (0, 0)),
      ],
      out_specs=[
          pl.BlockSpec((1, EBN, O), lambda b, i: (b, i, 0)),
          pl.BlockSpec((1, 8, O), lambda b, i: (b, 0, 0)),
      ],
      out_shape=[
          jax.ShapeDtypeStruct((B, L, O), jnp.float32),
          jax.ShapeDtypeStruct((B, 8, O), jnp.float32),
      ],
      scratch_shapes=[
          pltpu.VMEM((EBN, L), jnp.float32),
          pltpu.VMEM((EBN, O), jnp.float32),
          pltpu.VMEM((EBN, O), jnp.float32),
          pltpu.VMEM((EBN, O), jnp.float32),
      ],
      compiler_params=_CP,
      interpret=_INTERPRET,
  )(x, wt)


def _edge_layer_fact(x, wat, wdt, B, L, O):
  C = x.shape[-1]
  return pl.pallas_call(
      functools.partial(_edge_fact_kernel, L=L, O=O),
      grid=(B, L // EBN),
      in_specs=[
          pl.BlockSpec((1, L, C), lambda b, i: (b, 0, 0)),
          pl.BlockSpec((C, O), lambda b, i: (0, 0)),
          pl.BlockSpec((C, O), lambda b, i: (0, 0)),
      ],
      out_specs=[
          pl.BlockSpec((1, EBN, O), lambda b, i: (b, i, 0)),
          pl.BlockSpec((1, 8, O), lambda b, i: (b, 0, 0)),
      ],
      out_shape=[
          jax.ShapeDtypeStruct((B, L, O), jnp.float32),
          jax.ShapeDtypeStruct((B, 8, O), jnp.float32),
      ],
      scratch_shapes=[
          pltpu.VMEM((L, O), jnp.float32),
          pltpu.VMEM((L, O), jnp.float32),
          pltpu.VMEM((EBN, L), jnp.float32),
          pltpu.VMEM((EBN, O), jnp.float32),
          pltpu.VMEM((EBN, O), jnp.float32),
          pltpu.VMEM((EBN, O), jnp.float32),
      ],
      compiler_params=_CP,
      interpret=_INTERPRET,
  )(x, wat, wdt)


def _finalize_layer(ymax, part, gb, emb, B, L, O, E, fact):
  cnt = float(B * L * KNN)
  return pl.pallas_call(
      functools.partial(_finalize_kernel, cnt=cnt, O=O, fact=fact),
      grid=(B, L // BN),
      in_specs=[
          pl.BlockSpec((1, BN, O), lambda b, i: (b, i, 0)),
          pl.BlockSpec((B, 8, O), lambda b, i: (0, 0, 0)),
          pl.BlockSpec((2, O), lambda b, i: (0, 0)),
          pl.BlockSpec((1, BN, E), lambda b, i: (b, i, 0)),
      ],
      out_specs=pl.BlockSpec((1, BN, O + E), lambda b, i: (b, i, 0)),
      out_shape=jax.ShapeDtypeStruct((B, L, O + E), jnp.float32),
      compiler_params=_CP,
      interpret=_INTERPRET,
  )(ymax, part, gb, emb)


def kernel(occupy, level, octant, pos, e0_32, e1_32, e2_32, e0_128, e1_128,
           e2_128, e0_512, e1_512, e2_512, W1, g1, b1, W3, g3, b3, W5, g5,
           b5, Wc, gc, bc, Wm, bm):
  L, B, _ = pos.shape
  nb = L // BN
  f32 = jnp.float32

  occT = jnp.transpose(occupy, (1, 0, 2)).astype(jnp.int32)
  levT = jnp.transpose(level, (1, 0, 2)).astype(jnp.int32)
  octT = jnp.transpose(octant, (1, 0, 2)).astype(jnp.int32)
  xpos = jnp.transpose(pos, (1, 0, 2))

  wa5 = W5[:, :256]
  wat5, wdt5 = wa5.T, (W5[:, 256:] - wa5).T
  gb1 = jnp.stack([g1, b1])
  gb3 = jnp.stack([g3, b3])
  gb5 = jnp.stack([g5, b5])
  gbc = jnp.stack([gc, bc])
  bm2 = bm[None, :]

  nidx = B * L * 4
  occF = occT.reshape(1, nidx)
  levF = levT.reshape(1, nidx)
  octF = octT.reshape(1, nidx)
  tf32 = _fuse_tables(e0_32, e1_32, e2_32)
  tf128 = _fuse_tables(e0_128, e1_128, e2_128)
  tf512 = _fuse_tables(e0_512, e1_512, e2_512)
  g32, g128, g512 = _sc_emb_gather(occF, levF, octF, tf32, tf128, tf512, nidx)
  emb32 = g32.reshape(B, L, 4, 128)[..., :8].reshape(B, L, 32)
  emb128 = g128.reshape(B, L, 4, 128)[..., :32].reshape(B, L, 128)
  emb512 = g512.reshape(B, L, 512)

  ymax1, part1 = _edge_layer_exact(xpos, W1.T, B, L, 32)
  x1cat = _finalize_layer(ymax1, part1, gb1, emb32, B, L, 32, 32, False)

  ymax3, part3 = _edge_layer_exact(x1cat, W3.T, B, L, 128)
  x3cat = _finalize_layer(ymax3, part3, gb3, emb128, B, L, 128, 128, False)

  ymax5, part5 = _edge_layer_fact(x3cat, wat5, wdt5, B, L, 512)
  x5cat = _finalize_layer(ymax5, part5, gb5, emb512, B, L, 512, 512, True)

  ycat, partc = pl.pallas_call(
      _head_kernel,
      grid=(B, nb),
      in_specs=[
          pl.BlockSpec((1, BN, 64), lambda b, i: (b, i, 0)),
          pl.BlockSpec((1, BN, 256), lambda b, i: (b, i, 0)),
          pl.BlockSpec((1, BN, 1024), lambda b, i: (b, i, 0)),
          pl.BlockSpec((1344, 512), lambda b, i: (0, 0)),
      ],
      out_specs=[
          pl.BlockSpec((1, BN, 512), lambda b, i: (b, i, 0)),
          pl.BlockSpec((1, 8, 512), lambda b, i: (b, 0, 0)),
      ],
      out_shape=[
          jax.ShapeDtypeStruct((B, L, 512), f32),
          jax.ShapeDtypeStruct((B, 8, 512), f32),
      ],
      compiler_params=_CP,
      interpret=_INTERPRET,
  )(x1cat, x3cat, x5cat, Wc.T)

  pool = pl.pallas_call(
      functools.partial(_pool_kernel, cnt=float(B * L)),
      grid=(B, nb),
      in_specs=[
          pl.BlockSpec((1, BN, 512), lambda b, i: (b, i, 0)),
          pl.BlockSpec((B, 8, 512), lambda b, i: (0, 0, 0)),
          pl.BlockSpec((2, 512), lambda b, i: (0, 0)),
      ],
      out_specs=pl.BlockSpec((1, 8, 512), lambda b, i: (b, 0, 0)),
      out_shape=jax.ShapeDtypeStruct((B, 8, 512), f32),
      compiler_params=_CP,
      interpret=_INTERPRET,
  )(ycat, partc, gbc)

  outb = pl.pallas_call(
      functools.partial(_out_kernel, L=L),
      grid=(B, nb),
      in_specs=[
          pl.BlockSpec((1, BN, 1024), lambda b, i: (b, i, 0)),
          pl.BlockSpec((1, 8, 512), lambda b, i: (b, 0, 0)),
          pl.BlockSpec((2048, 512), lambda b, i: (0, 0)),
          pl.BlockSpec((1, 512), lambda b, i: (0, 0)),
      ],
      out_specs=pl.BlockSpec((1, BN, 512), lambda b, i: (b, i, 0)),
      out_shape=jax.ShapeDtypeStruct((B, L, 512), f32),
      compiler_params=_CP,
      interpret=_INTERPRET,
  )(x5cat, pool, Wm.T, bm2)

  return jnp.transpose(outb, (1, 0, 2))


# speedup vs baseline: 1.5051x; 1.5051x over previous
import functools

import jax
import jax.numpy as jnp
from jax import lax
from jax.experimental import pallas as pl
from jax.experimental.pallas import tpu as pltpu
from jax.experimental.pallas import tpu_sc as plsc

BN = 256
EBN = 1024
KNN = 20
_UNROLL = False
_INTERPRET = False

_NEG = float("-inf")
_HI = lax.Precision.HIGHEST


def _emb_kernel(occ_ref, lev_ref, oct_ref, t032, t132, t232, t0128, t1128,
                t2128, t0512, t1512, t2512, e32_ref, e128_ref, e512_ref):
  i257 = lax.broadcasted_iota(jnp.int32, (BN, 257), 1)
  i19 = lax.broadcasted_iota(jnp.int32, (BN, 19), 1)
  i9 = lax.broadcasted_iota(jnp.int32, (BN, 9), 1)
  f32 = jnp.float32
  for j in range(4):
    oc = occ_ref[0, :, j]
    oh = (oc[:, None] == i257).astype(f32)
    e32_ref[0, :, j * 8:j * 8 + 6] = jnp.dot(
        oh, t032[...], preferred_element_type=f32, precision=_HI)
    e128_ref[0, :, j * 32:j * 32 + 30] = jnp.dot(
        oh, t0128[...], preferred_element_type=f32, precision=_HI)
    e512_ref[0, :, j * 128:j * 128 + 126] = jnp.dot(
        oh, t0512[...], preferred_element_type=f32, precision=_HI)
    lv = lev_ref[0, :, j]
    ohl = (lv[:, None] == i19).astype(f32)
    e32_ref[0, :, j * 8 + 6:j * 8 + 7] = jnp.dot(
        ohl, t132[...], preferred_element_type=f32, precision=_HI)
    e128_ref[0, :, j * 32 + 30:j * 32 + 31] = jnp.dot(
        ohl, t1128[...], preferred_element_type=f32, precision=_HI)
    e512_ref[0, :, j * 128 + 126:j * 128 + 127] = jnp.dot(
        ohl, t1512[...], preferred_element_type=f32, precision=_HI)
    ot = oct_ref[0, :, j]
    oho = (ot[:, None] == i9).astype(f32)
    e32_ref[0, :, j * 8 + 7:j * 8 + 8] = jnp.dot(
        oho, t232[...], preferred_element_type=f32, precision=_HI)
    e128_ref[0, :, j * 32 + 31:j * 32 + 32] = jnp.dot(
        oho, t2128[...], preferred_element_type=f32, precision=_HI)
    e512_ref[0, :, j * 128 + 127:j * 128 + 128] = jnp.dot(
        oho, t2512[...], preferred_element_type=f32, precision=_HI)


_SC_W = 128


def _sc_mesh():
  return plsc.VectorSubcoreMesh(core_axis_name="c", subcore_axis_name="s")


def _sc_gather_rows(idxF, table, nidx):
  f32 = jnp.float32

  @functools.partial(
      pl.kernel,
      out_type=jax.ShapeDtypeStruct((nidx, 128), f32),
      mesh=_sc_mesh(),
  )
  def k(i_hbm, t_hbm, o_hbm):
    def body(i_v, o_v):
      pltpu.sync_copy(t_hbm.at[i_v.at[0]], o_v)

    pltpu.emit_pipeline(
        body,
        grid=(nidx // _SC_W,),
        in_specs=[pl.BlockSpec((1, _SC_W), lambda i: (0, i))],
        out_specs=[pl.BlockSpec((_SC_W, 128), lambda i: (i, 0))],
        core_axis_name=("c", "s"),
        dimension_semantics=(pltpu.PARALLEL,),
    )(i_hbm, o_hbm)

  return k(idxF, table)


def _sc_gather_scalar(levF, octF, tsc, nidx):
  f32 = jnp.float32

  @functools.partial(
      pl.kernel,
      out_type=jax.ShapeDtypeStruct((nidx, 128), f32),
      mesh=_sc_mesh(),
      scratch_types=[pltpu.VMEM((1, _SC_W), jnp.int32)],
  )
  def k(lev_hbm, oct_hbm, t_hbm, o_hbm, idx_s):
    def body(lev_v, oct_v, o_v):
      @pl.loop(0, _SC_W, step=16)
      def _(j):
        sl = (0, pl.ds(j, 16))
        idx_s[sl] = lev_v[sl] * 9 + oct_v[sl]

      pltpu.sync_copy(t_hbm.at[idx_s.at[0]], o_v)

    pltpu.emit_pipeline(
        body,
        grid=(nidx // _SC_W,),
        in_specs=[pl.BlockSpec((1, _SC_W), lambda i: (0, i))] * 2,
        out_specs=[pl.BlockSpec((_SC_W, 128), lambda i: (i, 0))],
        core_axis_name=("c", "s"),
        dimension_semantics=(pltpu.PARALLEL,),
    )(lev_hbm, oct_hbm, o_hbm)

  return k(levF, octF, tsc)


def _pairwise_block(x_ref, i, L, bn):
  f32 = jnp.float32
  x = x_ref[0]
  xblk = x_ref[0, pl.ds(i * bn, bn), :]
  g = lax.dot_general(xblk, x, (((1,), (1,)), ((), ())),
                      preferred_element_type=f32)
  xxall = jnp.sum(x * x, axis=1)
  xxblk = jnp.sum(xblk * xblk, axis=1)
  inner = -2.0 * g
  return (-xxblk[:, None]) - inner - xxall[None, :]


def _topk_passes(pd_s, body_fn, L, bn):
  iota = lax.broadcasted_iota(jnp.int32, (bn, L), 1)

  def one_pass(t, carry):
    pdv = pd_s[...]
    vmax = jnp.max(pdv, axis=1, keepdims=True)
    eq = pdv == vmax
    cand = jnp.where(eq, iota, L)
    pos = jnp.min(cand, axis=1, keepdims=True)
    h = iota == pos
    pd_s[...] = jnp.where(h, _NEG, pdv)
    body_fn(h.astype(jnp.bfloat16))
    return carry

  if _UNROLL:
    for t in range(KNN):
      one_pass(t, 0)
  else:
    lax.fori_loop(0, KNN, one_pass, 0, unroll=False)


def _edge_exact_kernel(x_ref, wt_ref, ymax_ref, part_ref,
                       pd_s, s_s, q_s, m_s, *, L, O, C):
  f32 = jnp.float32
  i = pl.program_id(1)

  @pl.when(i == 0)
  def _():
    part_ref[...] = jnp.zeros(part_ref.shape, f32)

  bf16 = jnp.bfloat16
  x = x_ref[0]
  xblk = x_ref[0, pl.ds(i * EBN, EBN), :]
  pd_s[...] = _pairwise_block(x_ref, i, L, EBN)
  s_s[...] = jnp.zeros((EBN, O), f32)
  q_s[...] = jnp.zeros((EBN, O), f32)
  m_s[...] = jnp.full((EBN, O), _NEG, f32)

  xh = x.astype(bf16)
  r1 = x - xh.astype(f32)
  xl = r1.astype(bf16)
  xll = (r1 - xl.astype(f32)).astype(bf16)

  def body(hb):
    feat = (jnp.dot(hb, xh, preferred_element_type=f32)
            + jnp.dot(hb, xl, preferred_element_type=f32)) \
        + jnp.dot(hb, xll, preferred_element_type=f32)
    v = jnp.concatenate([feat - xblk, xblk], axis=1)
    y = jnp.dot(v, wt_ref[...], preferred_element_type=f32)
    s_s[...] += y
    q_s[...] += y * y
    m_s[...] = jnp.maximum(m_s[...], y)

  _topk_passes(pd_s, body, L, EBN)

  ymax_ref[0] = m_s[...]
  zrow = jnp.zeros((1, O), f32)
  upd = jnp.concatenate([
      jnp.sum(s_s[...], axis=0)[None, :],
      jnp.sum(q_s[...], axis=0)[None, :],
      zrow, zrow, zrow, zrow, zrow, zrow], axis=0)
  part_ref[0] += upd


def _edge_fact_kernel(x_ref, wat_ref, wdt_ref, ymax_ref, part_ref,
                      z_s, w_s, pd_s, s_s, q_s, m_s, *, L, O):
  f32 = jnp.float32
  i = pl.program_id(1)

  @pl.when(i == 0)
  def _():
    x0 = x_ref[0]
    z_s[...] = jnp.dot(
        x0, wat_ref[...], preferred_element_type=f32).astype(jnp.bfloat16)
    w_s[...] = jnp.dot(x0, wdt_ref[...], preferred_element_type=f32)
    part_ref[...] = jnp.zeros(part_ref.shape, f32)

  pd_s[...] = _pairwise_block(x_ref, i, L, EBN)
  s_s[...] = jnp.zeros((EBN, O), f32)
  q_s[...] = jnp.zeros((EBN, O), f32)
  m_s[...] = jnp.full((EBN, O), _NEG, f32)

  def body(hb):
    gi = jnp.dot(hb, z_s[...], preferred_element_type=f32)
    s_s[...] += gi
    q_s[...] += gi * gi
    m_s[...] = jnp.maximum(m_s[...], gi)

  _topk_passes(pd_s, body, L, EBN)

  wblk = w_s[pl.ds(i * EBN, EBN), :]
  sv = s_s[...]
  ymax_ref[0] = m_s[...] + wblk
  zrow = jnp.zeros((1, O), f32)
  upd = jnp.concatenate([
      jnp.sum(sv, axis=0)[None, :],
      jnp.sum(q_s[...], axis=0)[None, :],
      jnp.sum(wblk * sv, axis=0)[None, :],
      jnp.sum(wblk, axis=0)[None, :],
      jnp.sum(wblk * wblk, axis=0)[None, :],
      zrow, zrow, zrow], axis=0)
  part_ref[0] += upd


def _finalize_kernel(ymax_ref, part_ref, gb_ref, emb_ref, xcat_ref, *,
                     cnt, O, fact):
  ps = part_ref[0] + part_ref[1]
  if fact:
    mu = (ps[0] + float(KNN) * ps[3]) / cnt
    e2 = (ps[1] + 2.0 * ps[2] + float(KNN) * ps[4]) / cnt
  else:
    mu = ps[0] / cnt
    e2 = ps[1] / cnt
  var = e2 - mu * mu
  sq = jnp.sqrt(var + 1e-5)
  v = (ymax_ref[0] - mu[None, :]) / sq[None, :] * gb_ref[0][None, :] \
      + gb_ref[1][None, :]
  v = jnp.where(v >= 0, v, 0.2 * v)
  xcat_ref[0, :, :O] = v
  xcat_ref[0, :, O:] = emb_ref[0]


def _head_kernel(x1_ref, x3_ref, x5_ref, wct_ref, y_ref, part_ref):
  f32 = jnp.float32
  i = pl.program_id(1)

  @pl.when(i == 0)
  def _():
    part_ref[...] = jnp.zeros(part_ref.shape, f32)

  xcat = jnp.concatenate([x1_ref[0], x3_ref[0], x5_ref[0]], axis=1)
  y = jnp.dot(xcat, wct_ref[...], preferred_element_type=f32)
  y_ref[0] = y
  zrow = jnp.zeros((1, 512), f32)
  upd = jnp.concatenate([
      jnp.sum(y, axis=0)[None, :],
      jnp.sum(y * y, axis=0)[None, :],
      zrow, zrow, zrow, zrow, zrow, zrow], axis=0)
  part_ref[0] += upd


def _pool_kernel(y_ref, part_ref, gb_ref, pool_ref, *, cnt):
  i = pl.program_id(1)
  ps = part_ref[0] + part_ref[1]
  mu = ps[0] / cnt
  var = ps[1] / cnt - mu * mu
  sq = jnp.sqrt(var + 1e-5)
  yn = (y_ref[0] - mu[None, :]) / sq[None, :] * gb_ref[0][None, :] \
      + gb_ref[1][None, :]
  yn = jnp.where(yn >= 0, yn, 0.2 * yn)

  @pl.when(i == 0)
  def _():
    pool_ref[...] = jnp.full(pool_ref.shape, _NEG, jnp.float32)
    pool_ref[0, 0, :] = jnp.zeros((512,), jnp.float32)

  pool_ref[0, 0, :] += jnp.sum(yn, axis=0)
  pool_ref[0, 1, :] = jnp.maximum(pool_ref[0, 1, :], jnp.max(yn, axis=0))


def _out_kernel(x5_ref, pool_ref, wmt_ref, bm_ref, o_ref, *, L):
  f32 = jnp.float32
  avg = pool_ref[0, 0, :] / float(L)
  mx = pool_ref[0, 1, :]
  big = jnp.concatenate([
      x5_ref[0],
      jnp.broadcast_to(avg[None, :], (BN, 512)),
      jnp.broadcast_to(mx[None, :], (BN, 512))], axis=1)
  o_ref[0] = jnp.dot(big, wmt_ref[...], preferred_element_type=f32) \
      + bm_ref[...]


_CP = pltpu.CompilerParams(dimension_semantics=("parallel", "arbitrary"))


def _edge_layer_exact(x, wt, B, L, O):
  C = x.shape[-1]
  return pl.pallas_call(
      functools.partial(_edge_exact_kernel, L=L, O=O, C=C),
      grid=(B, L // EBN),
      in_specs=[
          pl.BlockSpec((1, L, C), lambda b, i: (b, 0, 0)),
          pl.BlockSpec((2 * C, O), lambda b, i: (0, 0)),
      ],
      out_specs=[
          pl.BlockSpec((1, EBN, O), lambda b, i: (b, i, 0)),
          pl.BlockSpec((1, 8, O), lambda b, i: (b, 0, 0)),
      ],
      out_shape=[
          jax.ShapeDtypeStruct((B, L, O), jnp.float32),
          jax.ShapeDtypeStruct((B, 8, O), jnp.float32),
      ],
      scratch_shapes=[
          pltpu.VMEM((EBN, L), jnp.float32),
          pltpu.VMEM((EBN, O), jnp.float32),
          pltpu.VMEM((EBN, O), jnp.float32),
          pltpu.VMEM((EBN, O), jnp.float32),
      ],
      compiler_params=_CP,
      interpret=_INTERPRET,
  )(x, wt)


def _edge_layer_fact(x, wat, wdt, B, L, O):
  C = x.shape[-1]
  return pl.pallas_call(
      functools.partial(_edge_fact_kernel, L=L, O=O),
      grid=(B, L // EBN),
      in_specs=[
          pl.BlockSpec((1, L, C), lambda b, i: (b, 0, 0)),
          pl.BlockSpec((C, O), lambda b, i: (0, 0)),
          pl.BlockSpec((C, O), lambda b, i: (0, 0)),
      ],
      out_specs=[
          pl.BlockSpec((1, EBN, O), lambda b, i: (b, i, 0)),
          pl.BlockSpec((1, 8, O), lambda b, i: (b, 0, 0)),
      ],
      out_shape=[
          jax.ShapeDtypeStruct((B, L, O), jnp.float32),
          jax.ShapeDtypeStruct((B, 8, O), jnp.float32),
      ],
      scratch_shapes=[
          pltpu.VMEM((L, O), jnp.bfloat16),
          pltpu.VMEM((L, O), jnp.float32),
          pltpu.VMEM((EBN, L), jnp.float32),
          pltpu.VMEM((EBN, O), jnp.float32),
          pltpu.VMEM((EBN, O), jnp.float32),
          pltpu.VMEM((EBN, O), jnp.float32),
      ],
      compiler_params=_CP,
      interpret=_INTERPRET,
  )(x, wat, wdt)


def _finalize_layer(ymax, part, gb, emb, B, L, O, E, fact):
  cnt = float(B * L * KNN)
  return pl.pallas_call(
      functools.partial(_finalize_kernel, cnt=cnt, O=O, fact=fact),
      grid=(B, L // BN),
      in_specs=[
          pl.BlockSpec((1, BN, O), lambda b, i: (b, i, 0)),
          pl.BlockSpec((B, 8, O), lambda b, i: (0, 0, 0)),
          pl.BlockSpec((2, O), lambda b, i: (0, 0)),
          pl.BlockSpec((1, BN, E), lambda b, i: (b, i, 0)),
      ],
      out_specs=pl.BlockSpec((1, BN, O + E), lambda b, i: (b, i, 0)),
      out_shape=jax.ShapeDtypeStruct((B, L, O + E), jnp.float32),
      compiler_params=_CP,
      interpret=_INTERPRET,
  )(ymax, part, gb, emb)


def kernel(occupy, level, octant, pos, e0_32, e1_32, e2_32, e0_128, e1_128,
           e2_128, e0_512, e1_512, e2_512, W1, g1, b1, W3, g3, b3, W5, g5,
           b5, Wc, gc, bc, Wm, bm):
  L, B, _ = pos.shape
  nb = L // BN
  f32 = jnp.float32

  occT = jnp.transpose(occupy, (1, 0, 2)).astype(jnp.int32)
  levT = jnp.transpose(level, (1, 0, 2)).astype(jnp.int32)
  octT = jnp.transpose(octant, (1, 0, 2)).astype(jnp.int32)
  xpos = jnp.transpose(pos, (1, 0, 2))

  wa5 = W5[:, :256]
  wat5, wdt5 = wa5.T, (W5[:, 256:] - wa5).T
  gb1 = jnp.stack([g1, b1])
  gb3 = jnp.stack([g3, b3])
  gb5 = jnp.stack([g5, b5])
  gbc = jnp.stack([gc, bc])
  bm2 = bm[None, :]

  nidx = B * L * 4
  occF = occT.reshape(1, nidx)
  levF = levT.reshape(1, nidx)
  octF = octT.reshape(1, nidx)
  t032 = jnp.pad(e0_32, ((0, 0), (0, 122)))
  t0128 = jnp.pad(e0_128, ((0, 0), (0, 98)))
  t0512 = jnp.pad(e0_512, ((0, 0), (0, 2)))
  tsc = jnp.pad(
      jnp.concatenate([
          jnp.repeat(e1_32, 9, axis=0), jnp.tile(e2_32, (19, 1)),
          jnp.repeat(e1_128, 9, axis=0), jnp.tile(e2_128, (19, 1)),
          jnp.repeat(e1_512, 9, axis=0), jnp.tile(e2_512, (19, 1)),
      ], axis=1), ((0, 0), (0, 122)))
  g32 = _sc_gather_rows(occF, t032, nidx)
  g128 = _sc_gather_rows(occF, t0128, nidx)
  g512 = _sc_gather_rows(occF, t0512, nidx)
  gsc = _sc_gather_scalar(levF, octF, tsc, nidx)
  emb32 = jnp.concatenate(
      [g32[:, :6].reshape(B, L, 4, 6), gsc[:, 0:2].reshape(B, L, 4, 2)],
      axis=-1).reshape(B, L, 32)
  emb128 = jnp.concatenate(
      [g128[:, :30].reshape(B, L, 4, 30), gsc[:, 2:4].reshape(B, L, 4, 2)],
      axis=-1).reshape(B, L, 128)
  emb512 = jnp.concatenate(
      [g512[:, :126].reshape(B, L, 4, 126), gsc[:, 4:6].reshape(B, L, 4, 2)],
      axis=-1).reshape(B, L, 512)

  ymax1, part1 = _edge_layer_exact(xpos, W1.T, B, L, 32)
  x1cat = _finalize_layer(ymax1, part1, gb1, emb32, B, L, 32, 32, False)

  ymax3, part3 = _edge_layer_exact(x1cat, W3.T, B, L, 128)
  x3cat = _finalize_layer(ymax3, part3, gb3, emb128, B, L, 128, 128, False)

  ymax5, part5 = _edge_layer_fact(x3cat, wat5, wdt5, B, L, 512)
  x5cat = _finalize_layer(ymax5, part5, gb5, emb512, B, L, 512, 512, True)

  ycat, partc = pl.pallas_call(
      _head_kernel,
      grid=(B, nb),
      in_specs=[
          pl.BlockSpec((1, BN, 64), lambda b, i: (b, i, 0)),
          pl.BlockSpec((1, BN, 256), lambda b, i: (b, i, 0)),
          pl.BlockSpec((1, BN, 1024), lambda b, i: (b, i, 0)),
          pl.BlockSpec((1344, 512), lambda b, i: (0, 0)),
      ],
      out_specs=[
          pl.BlockSpec((1, BN, 512), lambda b, i: (b, i, 0)),
          pl.BlockSpec((1, 8, 512), lambda b, i: (b, 0, 0)),
      ],
      out_shape=[
          jax.ShapeDtypeStruct((B, L, 512), f32),
          jax.ShapeDtypeStruct((B, 8, 512), f32),
      ],
      compiler_params=_CP,
      interpret=_INTERPRET,
  )(x1cat, x3cat, x5cat, Wc.T)

  pool = pl.pallas_call(
      functools.partial(_pool_kernel, cnt=float(B * L)),
      grid=(B, nb),
      in_specs=[
          pl.BlockSpec((1, BN, 512), lambda b, i: (b, i, 0)),
          pl.BlockSpec((B, 8, 512), lambda b, i: (0, 0, 0)),
          pl.BlockSpec((2, 512), lambda b, i: (0, 0)),
      ],
      out_specs=pl.BlockSpec((1, 8, 512), lambda b, i: (b, 0, 0)),
      out_shape=jax.ShapeDtypeStruct((B, 8, 512), f32),
      compiler_params=_CP,
      interpret=_INTERPRET,
  )(ycat, partc, gbc)

  outb = pl.pallas_call(
      functools.partial(_out_kernel, L=L),
      grid=(B, nb),
      in_specs=[
          pl.BlockSpec((1, BN, 1024), lambda b, i: (b, i, 0)),
          pl.BlockSpec((1, 8, 512), lambda b, i: (b, 0, 0)),
          pl.BlockSpec((2048, 512), lambda b, i: (0, 0)),
          pl.BlockSpec((1, 512), lambda b, i: (0, 0)),
      ],
      out_specs=pl.BlockSpec((1, BN, 512), lambda b, i: (b, i, 0)),
      out_shape=jax.ShapeDtypeStruct((B, L, 512), f32),
      compiler_params=_CP,
      interpret=_INTERPRET,
  )(x5cat, pool, Wm.T, bm2)

  return jnp.transpose(outb, (1, 0, 2))
